# baseline (speedup 1.0000x reference)
import functools

import jax
import jax.numpy as jnp
from jax import lax
from jax.experimental import pallas as pl
from jax.experimental.pallas import tpu as pltpu
from jax.experimental.pallas import tpu_sc as plsc

B = 16
C = 3
H = 256
W = 256
OUT_H = 128
OUT_W = 64
NNZ = 98304
OUT_DIM = OUT_H * OUT_W * C
IN_DIM = H * W * C
L = 16

CHUNK = 8192


def _floordiv3(v):
    return (v.astype(jnp.float32) * jnp.float32(1.0 / 3.0)).astype(jnp.int32)


def _sc_spmv(x1d, rows, cols, vals, mask1d):
    info = plsc.get_sparse_core_info()
    num_cores, num_subcores = info.num_cores, info.num_subcores
    items_per_core = B // num_cores
    tiles_per_item = num_subcores // items_per_core
    nnz_per_tile = NNZ // tiles_per_item
    n_chunks = nnz_per_tile // CHUNK
    out_slice = OUT_DIM // tiles_per_item

    mesh = plsc.VectorSubcoreMesh(core_axis_name="c", subcore_axis_name="s")

    @functools.partial(
        pl.kernel,
        out_type=[jax.ShapeDtypeStruct((B * OUT_DIM,), jnp.float32),
                  jax.ShapeDtypeStruct((B * OUT_DIM,), jnp.float32)],
        mesh=mesh,
        scratch_types=(
            [pltpu.VMEM_SHARED((items_per_core * OUT_DIM,), jnp.float32)]
            + [pltpu.VMEM((CHUNK,), jnp.int32) for _ in range(2)]
            + [pltpu.VMEM((CHUNK,), jnp.int32) for _ in range(2)]
            + [pltpu.VMEM((CHUNK,), jnp.float32) for _ in range(2)]
            + [pltpu.VMEM((CHUNK,), jnp.int32) for _ in range(2)]
            + [pltpu.VMEM((CHUNK,), jnp.int32) for _ in range(2)]
            + [pltpu.VMEM((CHUNK,), jnp.float32) for _ in range(2)]
            + [pltpu.VMEM((CHUNK,), jnp.int32)]
            + [pltpu.VMEM((out_slice - CHUNK,), jnp.int32)]
            + [pltpu.VMEM((out_slice - CHUNK,), jnp.float32)]
            + [pltpu.SemaphoreType.DMA for _ in range(5)]
        ),
    )
    def run(x_hbm, rows_hbm, cols_hbm, vals_hbm, mask_hbm, out_hbm, masks_hbm,
            shared_acc, cols_v0, cols_v1, rows_v0, rows_v1,
            vals_v0, vals_v1, gidx_v0, gidx_v1, sidx_v0, sidx_v1,
            xv_v0, xv_v1, midx_a, midx_b, mstage_b,
            sem_in0, sem_in1, sem_g, sem_s, sem_m):
        cols_v = (cols_v0, cols_v1)
        rows_v = (rows_v0, rows_v1)
        vals_v = (vals_v0, vals_v1)
        gidx_v = (gidx_v0, gidx_v1)
        sidx_v = (sidx_v0, sidx_v1)
        xv_v = (xv_v0, xv_v1)
        cid = lax.axis_index("c")
        sid = lax.axis_index("s")
        slot = sid // tiles_per_item
        half = sid % tiles_per_item
        item = cid * items_per_core + slot

        gbase = item * IN_DIM
        sbase = slot * OUT_DIM
        nnz_base = half * nnz_per_tile
        sem_in = (sem_in0, sem_in1)

        def dma_in(ci):
            b = ci % 2
            base = nnz_base + ci * CHUNK
            sem = sem_in[b]
            return (
                pltpu.async_copy(cols_hbm.at[item, pl.ds(base, CHUNK)],
                                 cols_v[b], sem),
                pltpu.async_copy(rows_hbm.at[item, pl.ds(base, CHUNK)],
                                 rows_v[b], sem),
                pltpu.async_copy(vals_hbm.at[item, pl.ds(base, CHUNK)],
                                 vals_v[b], sem),
            )

        in_pend = [dma_in(0), dma_in(1)]

        @plsc.parallel_loop(0, CHUNK, step=L, unroll=8)
        def zero_body(i):
            xv_v0[pl.ds(i, L)] = jnp.zeros((L,), jnp.float32)
        z_left = out_slice
        z_off = slot * OUT_DIM + half * out_slice
        while z_left > 0:
            z = min(z_left, CHUNK)
            pltpu.sync_copy(xv_v0.at[pl.ds(0, z)],
                            shared_acc.at[pl.ds(z_off, z)])
            z_off += z
            z_left -= z

        plsc.subcore_barrier()

        def remap(b, i):
            sl = pl.ds(i, L)
            cc = cols_v[b][sl]
            q = _floordiv3(cc)
            gidx_v[b][sl] = cc * (H * W) - q * (3 * H * W - 1) + gbase
            rr = rows_v[b][sl]
            q2 = _floordiv3(rr)
            sidx_v[b][sl] = (
                rr * (OUT_H * OUT_W) - q2 * (3 * OUT_H * OUT_W - 1) + sbase)

        for d in in_pend[0]:
            d.wait()

        @plsc.parallel_loop(0, CHUNK, step=L, unroll=8)
        def first_remap(i):
            remap(0, i)

        gat = pltpu.async_copy(x_hbm.at[gidx_v[0]], xv_v[0], sem_g)
        scat_prev = None
        for ci in range(n_chunks):
            b = ci % 2
            nb = 1 - b
            have_next = ci + 1 < n_chunks
            if have_next:
                for d in in_pend[1]:
                    d.wait()
            gat.wait()
            if scat_prev is not None:
                scat_prev.wait()

            if have_next:
                @plsc.parallel_loop(0, CHUNK, step=L, unroll=8)
                def fused_body(i):
                    sl = pl.ds(i, L)
                    xv_v[b][sl] = xv_v[b][sl] * vals_v[b][sl]
                    remap(nb, i)
                gat = pltpu.async_copy(x_hbm.at[gidx_v[nb]], xv_v[nb], sem_g)
            else:
                @plsc.parallel_loop(0, CHUNK, step=L, unroll=8)
                def tail_body(i):
                    sl = pl.ds(i, L)
                    xv_v[b][sl] = xv_v[b][sl] * vals_v[b][sl]

            in_pend = [in_pend[1], None]
            if ci + 2 < n_chunks:
                in_pend[1] = dma_in(ci + 2)

            scat_prev = pltpu.async_copy(
                xv_v[b], shared_acc.at[sidx_v[b]], sem_s, add=True)

        iota = lax.iota(jnp.int32, L)
        pbase = half * out_slice
        hw = OUT_H * OUT_W
        moff = item * OUT_DIM

        @plsc.parallel_loop(0, CHUNK, step=L, unroll=8)
        def mask_a(i):
            p = pbase + i + iota
            midx_a[pl.ds(i, L)] = (p & (hw - 1)) * 3 + (p >> 13) + moff

        @plsc.parallel_loop(0, out_slice - CHUNK, step=L, unroll=8)
        def mask_b(i):
            p = pbase + CHUNK + i + iota
            midx_b[pl.ds(i, L)] = (p & (hw - 1)) * 3 + (p >> 13) + moff

        mga = pltpu.async_copy(mask_hbm.at[midx_a], vals_v0, sem_m)
        mgb = pltpu.async_copy(mask_hbm.at[midx_b], mstage_b, sem_m)
        mga.wait()
        mgb.wait()
        mbase = item * OUT_DIM + pbase
        outa = pltpu.async_copy(vals_v0, masks_hbm.at[pl.ds(mbase, CHUNK)],
                                sem_m)
        outb = pltpu.async_copy(
            mstage_b, masks_hbm.at[pl.ds(mbase + CHUNK, out_slice - CHUNK)],
            sem_m)
        outa.wait()
        outb.wait()

        scat_prev.wait()

        plsc.subcore_barrier()

        pltpu.sync_copy(
            shared_acc.at[pl.ds(slot * OUT_DIM + half * out_slice, out_slice)],
            out_hbm.at[pl.ds(item * OUT_DIM + half * out_slice, out_slice)])

    return run(x1d, rows, cols, vals, mask1d)


def kernel(x, rows, cols, vals, mask):
    x1d = x.reshape(B * IN_DIM)
    out, masks_flat = _sc_spmv(x1d, rows, cols, vals, mask.reshape(B * OUT_DIM))
    result = out.reshape(B, C, OUT_H, OUT_W)
    masks = masks_flat.reshape(B, C, OUT_H, OUT_W)
    return (result, masks)


# speedup vs baseline: 1.8642x; 1.8642x over previous
import functools

import jax
import jax.numpy as jnp
from jax import lax
from jax.experimental import pallas as pl
from jax.experimental.pallas import tpu as pltpu
from jax.experimental.pallas import tpu_sc as plsc

B = 16
C = 3
H = 256
W = 256
OUT_H = 128
OUT_W = 64
NNZ = 98304
OUT_DIM = OUT_H * OUT_W * C
IN_DIM = H * W * C
L = 16

CHUNK = 8192


def _floordiv3(v):
    return (v.astype(jnp.float32) * jnp.float32(1.0 / 3.0)).astype(jnp.int32)


def _sc_spmv(x1d, rows, cols, vals):
    info = plsc.get_sparse_core_info()
    num_cores, num_subcores = info.num_cores, info.num_subcores
    items_per_core = B // num_cores
    tiles_per_item = num_subcores // items_per_core
    nnz_per_tile = NNZ // tiles_per_item
    n_chunks = nnz_per_tile // CHUNK
    out_slice = OUT_DIM // tiles_per_item

    mesh = plsc.VectorSubcoreMesh(core_axis_name="c", subcore_axis_name="s")

    @functools.partial(
        pl.kernel,
        out_type=jax.ShapeDtypeStruct((B * OUT_DIM,), jnp.float32),
        mesh=mesh,
        scratch_types=(
            [pltpu.VMEM_SHARED((items_per_core * OUT_DIM,), jnp.float32)]
            + [pltpu.VMEM((CHUNK,), jnp.int32) for _ in range(2)]
            + [pltpu.VMEM((CHUNK,), jnp.int32) for _ in range(2)]
            + [pltpu.VMEM((CHUNK,), jnp.float32) for _ in range(2)]
            + [pltpu.VMEM((CHUNK,), jnp.int32) for _ in range(2)]
            + [pltpu.VMEM((CHUNK,), jnp.int32) for _ in range(2)]
            + [pltpu.VMEM((CHUNK,), jnp.float32) for _ in range(2)]
            + [pltpu.SemaphoreType.DMA for _ in range(4)]
        ),
    )
    def run(x_hbm, rows_hbm, cols_hbm, vals_hbm, out_hbm,
            shared_acc, cols_v0, cols_v1, rows_v0, rows_v1,
            vals_v0, vals_v1, gidx_v0, gidx_v1, sidx_v0, sidx_v1,
            xv_v0, xv_v1, sem_in0, sem_in1, sem_g, sem_s):
        cols_v = (cols_v0, cols_v1)
        rows_v = (rows_v0, rows_v1)
        vals_v = (vals_v0, vals_v1)
        gidx_v = (gidx_v0, gidx_v1)
        sidx_v = (sidx_v0, sidx_v1)
        xv_v = (xv_v0, xv_v1)
        cid = lax.axis_index("c")
        sid = lax.axis_index("s")
        slot = sid // tiles_per_item
        half = sid % tiles_per_item
        item = cid * items_per_core + slot

        gbase = item * IN_DIM
        sbase = slot * OUT_DIM
        nnz_base = half * nnz_per_tile
        sem_in = (sem_in0, sem_in1)

        def dma_in(ci):
            b = ci % 2
            base = nnz_base + ci * CHUNK
            sem = sem_in[b]
            return (
                pltpu.async_copy(cols_hbm.at[item, pl.ds(base, CHUNK)],
                                 cols_v[b], sem),
                pltpu.async_copy(rows_hbm.at[item, pl.ds(base, CHUNK)],
                                 rows_v[b], sem),
                pltpu.async_copy(vals_hbm.at[item, pl.ds(base, CHUNK)],
                                 vals_v[b], sem),
            )

        in_pend = [dma_in(0), dma_in(1)]

        @plsc.parallel_loop(0, CHUNK, step=L, unroll=8)
        def zero_body(i):
            xv_v0[pl.ds(i, L)] = jnp.zeros((L,), jnp.float32)
        z_left = out_slice
        z_off = slot * OUT_DIM + half * out_slice
        while z_left > 0:
            z = min(z_left, CHUNK)
            pltpu.sync_copy(xv_v0.at[pl.ds(0, z)],
                            shared_acc.at[pl.ds(z_off, z)])
            z_off += z
            z_left -= z

        plsc.subcore_barrier()

        def remap(b, i):
            sl = pl.ds(i, L)
            cc = cols_v[b][sl]
            q = _floordiv3(cc)
            gidx_v[b][sl] = cc * (H * W) - q * (3 * H * W - 1) + gbase
            rr = rows_v[b][sl]
            q2 = _floordiv3(rr)
            sidx_v[b][sl] = (
                rr * (OUT_H * OUT_W) - q2 * (3 * OUT_H * OUT_W - 1) + sbase)

        for d in in_pend[0]:
            d.wait()

        @plsc.parallel_loop(0, CHUNK, step=L, unroll=8)
        def first_remap(i):
            remap(0, i)

        gat = pltpu.async_copy(x_hbm.at[gidx_v[0]], xv_v[0], sem_g)
        scat_prev = None
        for ci in range(n_chunks):
            b = ci % 2
            nb = 1 - b
            have_next = ci + 1 < n_chunks
            if have_next:
                for d in in_pend[1]:
                    d.wait()
            gat.wait()
            if scat_prev is not None:
                scat_prev.wait()

            if have_next:
                @plsc.parallel_loop(0, CHUNK, step=L, unroll=8)
                def fused_body(i):
                    sl = pl.ds(i, L)
                    xv_v[b][sl] = xv_v[b][sl] * vals_v[b][sl]
                    remap(nb, i)
                gat = pltpu.async_copy(x_hbm.at[gidx_v[nb]], xv_v[nb], sem_g)
            else:
                @plsc.parallel_loop(0, CHUNK, step=L, unroll=8)
                def tail_body(i):
                    sl = pl.ds(i, L)
                    xv_v[b][sl] = xv_v[b][sl] * vals_v[b][sl]

            in_pend = [in_pend[1], None]
            if ci + 2 < n_chunks:
                in_pend[1] = dma_in(ci + 2)

            scat_prev = pltpu.async_copy(
                xv_v[b], shared_acc.at[sidx_v[b]], sem_s, add=True)

        scat_prev.wait()

        plsc.subcore_barrier()

        pltpu.sync_copy(
            shared_acc.at[pl.ds(slot * OUT_DIM + half * out_slice, out_slice)],
            out_hbm.at[pl.ds(item * OUT_DIM + half * out_slice, out_slice)])

    return run(x1d, rows, cols, vals)


def kernel(x, rows, cols, vals, mask):
    x1d = x.reshape(B * IN_DIM)
    out = _sc_spmv(x1d, rows, cols, vals)
    result = out.reshape(B, C, OUT_H, OUT_W)
    masks = jnp.transpose(mask, (0, 3, 1, 2))
    return (result, masks)


# speedup vs baseline: 1.8714x; 1.0039x over previous
import functools

import jax
import jax.numpy as jnp
from jax import lax
from jax.experimental import pallas as pl
from jax.experimental.pallas import tpu as pltpu
from jax.experimental.pallas import tpu_sc as plsc

B = 16
C = 3
H = 256
W = 256
OUT_H = 128
OUT_W = 64
NNZ = 98304
OUT_DIM = OUT_H * OUT_W * C
IN_DIM = H * W * C
L = 16

CHUNK = 8192


def _floordiv3(v):
    return (v.astype(jnp.float32) * jnp.float32(1.0 / 3.0)).astype(jnp.int32)


def _sc_spmv(x1d, rows, cols, vals):
    info = plsc.get_sparse_core_info()
    num_cores, num_subcores = info.num_cores, info.num_subcores
    items_per_core = B // num_cores
    tiles_per_item = num_subcores // items_per_core
    nnz_per_tile = NNZ // tiles_per_item
    n_chunks = nnz_per_tile // CHUNK
    out_slice = OUT_DIM // tiles_per_item

    mesh = plsc.VectorSubcoreMesh(core_axis_name="c", subcore_axis_name="s")

    @functools.partial(
        pl.kernel,
        out_type=jax.ShapeDtypeStruct((B * OUT_DIM,), jnp.float32),
        mesh=mesh,
        scratch_types=(
            [pltpu.VMEM_SHARED((items_per_core * OUT_DIM,), jnp.float32)]
            + [pltpu.VMEM((CHUNK,), jnp.int32) for _ in range(2)]
            + [pltpu.VMEM((CHUNK,), jnp.int32) for _ in range(2)]
            + [pltpu.VMEM((CHUNK,), jnp.float32) for _ in range(2)]
            + [pltpu.VMEM((CHUNK,), jnp.int32) for _ in range(2)]
            + [pltpu.VMEM((CHUNK,), jnp.int32) for _ in range(2)]
            + [pltpu.VMEM((CHUNK,), jnp.float32) for _ in range(2)]
            + [pltpu.SemaphoreType.DMA for _ in range(4)]
        ),
    )
    def run(x_hbm, rows_hbm, cols_hbm, vals_hbm, out_hbm,
            shared_acc, cols_v0, cols_v1, rows_v0, rows_v1,
            vals_v0, vals_v1, gidx_v0, gidx_v1, sidx_v0, sidx_v1,
            xv_v0, xv_v1, sem_in0, sem_in1, sem_g, sem_s):
        cols_v = (cols_v0, cols_v1)
        rows_v = (rows_v0, rows_v1)
        vals_v = (vals_v0, vals_v1)
        gidx_v = (gidx_v0, gidx_v1)
        sidx_v = (sidx_v0, sidx_v1)
        xv_v = (xv_v0, xv_v1)
        cid = lax.axis_index("c")
        sid = lax.axis_index("s")
        slot = sid // tiles_per_item
        half = sid % tiles_per_item
        item = cid * items_per_core + slot

        gbase = item * IN_DIM
        sbase = slot * OUT_DIM
        nnz_base = half * nnz_per_tile
        sem_in = (sem_in0, sem_in1)

        def dma_in(ci):
            b = ci % 2
            base = nnz_base + ci * CHUNK
            sem = sem_in[b]
            return (
                pltpu.async_copy(cols_hbm.at[item, pl.ds(base, CHUNK)],
                                 cols_v[b], sem),
                pltpu.async_copy(rows_hbm.at[item, pl.ds(base, CHUNK)],
                                 rows_v[b], sem),
                pltpu.async_copy(vals_hbm.at[item, pl.ds(base, CHUNK)],
                                 vals_v[b], sem),
            )

        in_pend = [dma_in(0), dma_in(1)]

        @plsc.parallel_loop(0, CHUNK, step=L, unroll=8)
        def zero_body(i):
            xv_v0[pl.ds(i, L)] = jnp.zeros((L,), jnp.float32)
        z_left = out_slice
        z_off = slot * OUT_DIM + half * out_slice
        while z_left > 0:
            z = min(z_left, CHUNK)
            pltpu.sync_copy(xv_v0.at[pl.ds(0, z)],
                            shared_acc.at[pl.ds(z_off, z)])
            z_off += z
            z_left -= z

        plsc.subcore_barrier()

        def remap(b, i):
            sl = pl.ds(i, L)
            cc = cols_v[b][sl]
            q = _floordiv3(cc)
            gidx_v[b][sl] = cc * (H * W) - q * (3 * H * W - 1) + gbase
            rr = rows_v[b][sl]
            q2 = _floordiv3(rr)
            sidx_v[b][sl] = (
                rr * (OUT_H * OUT_W) - q2 * (3 * OUT_H * OUT_W - 1) + sbase)

        for d in in_pend[0]:
            d.wait()

        @plsc.parallel_loop(0, CHUNK, step=L, unroll=8)
        def first_remap(i):
            remap(0, i)

        gat = pltpu.async_copy(x_hbm.at[gidx_v[0]], xv_v[0], sem_g)
        scat_prev = None
        for ci in range(n_chunks):
            b = ci % 2
            nb = 1 - b
            have_next = ci + 1 < n_chunks
            if have_next:
                for d in in_pend[1]:
                    d.wait()
            gat.wait()
            if scat_prev is not None:
                scat_prev.wait()

            if have_next:
                @plsc.parallel_loop(0, CHUNK, step=L, unroll=8)
                def fused_body(i):
                    sl = pl.ds(i, L)
                    xv_v[b][sl] = xv_v[b][sl] * vals_v[b][sl]
                    remap(nb, i)
                gat = pltpu.async_copy(x_hbm.at[gidx_v[nb]], xv_v[nb], sem_g)
            else:
                @plsc.parallel_loop(0, CHUNK, step=L, unroll=8)
                def tail_body(i):
                    sl = pl.ds(i, L)
                    xv_v[b][sl] = xv_v[b][sl] * vals_v[b][sl]

            in_pend = [in_pend[1], None]
            if ci + 2 < n_chunks:
                in_pend[1] = dma_in(ci + 2)

            scat_prev = pltpu.async_copy(
                xv_v[b], shared_acc.at[sidx_v[b]], sem_s, add=True)

        scat_prev.wait()

        plsc.subcore_barrier()

        pltpu.sync_copy(
            shared_acc.at[pl.ds(slot * OUT_DIM + half * out_slice, out_slice)],
            out_hbm.at[pl.ds(item * OUT_DIM + half * out_slice, out_slice)])

    return run(x1d, rows, cols, vals)


def kernel(x, rows, cols, vals, mask):
    x1d = x.reshape(B * IN_DIM)
    out = _sc_spmv(x1d, rows, cols, vals)
    result = out.reshape(B, C, OUT_H, OUT_W)
    zero = (x1d[0] - x1d[0]).astype(jnp.float32)
    masks = jnp.transpose(mask, (0, 3, 1, 2)) + zero
    return (result, masks)


# speedup vs baseline: 1.9592x; 1.0469x over previous
import functools

import jax
import jax.numpy as jnp
from jax import lax
from jax.experimental import pallas as pl
from jax.experimental.pallas import tpu as pltpu
from jax.experimental.pallas import tpu_sc as plsc

B = 16
C = 3
H = 256
W = 256
OUT_H = 128
OUT_W = 64
NNZ = 98304
OUT_DIM = OUT_H * OUT_W * C
IN_DIM = H * W * C
L = 16

CHUNK = 6144


def _floordiv3(v):
    return (v.astype(jnp.float32) * jnp.float32(1.0 / 3.0)).astype(jnp.int32)


def _sc_spmv(x1d, rows, cols, vals):
    info = plsc.get_sparse_core_info()
    num_cores, num_subcores = info.num_cores, info.num_subcores
    items_per_core = B // num_cores
    tiles_per_item = num_subcores // items_per_core
    nnz_per_tile = NNZ // tiles_per_item
    n_chunks = nnz_per_tile // CHUNK
    out_slice = OUT_DIM // tiles_per_item

    mesh = plsc.VectorSubcoreMesh(core_axis_name="c", subcore_axis_name="s")

    @functools.partial(
        pl.kernel,
        out_type=jax.ShapeDtypeStruct((B * OUT_DIM,), jnp.float32),
        mesh=mesh,
        scratch_types=(
            [pltpu.VMEM_SHARED((items_per_core * OUT_DIM,), jnp.float32)]
            + [pltpu.VMEM((CHUNK,), jnp.int32) for _ in range(2)]
            + [pltpu.VMEM((CHUNK,), jnp.int32) for _ in range(2)]
            + [pltpu.VMEM((CHUNK,), jnp.float32) for _ in range(3)]
            + [pltpu.VMEM((CHUNK,), jnp.int32) for _ in range(2)]
            + [pltpu.VMEM((CHUNK,), jnp.int32) for _ in range(3)]
            + [pltpu.VMEM((CHUNK,), jnp.float32) for _ in range(3)]
            + [pltpu.SemaphoreType.DMA for _ in range(4)]
        ),
    )
    def run(x_hbm, rows_hbm, cols_hbm, vals_hbm, out_hbm,
            shared_acc, cols_v0, cols_v1, rows_v0, rows_v1,
            vals_v0, vals_v1, vals_v2, gidx_v0, gidx_v1,
            sidx_v0, sidx_v1, sidx_v2, xv_v0, xv_v1, xv_v2,
            sem_cr, sem_v, sem_g, sem_s):
        cols_v = (cols_v0, cols_v1)
        rows_v = (rows_v0, rows_v1)
        vals_v = (vals_v0, vals_v1, vals_v2)
        gidx_v = (gidx_v0, gidx_v1)
        sidx_v = (sidx_v0, sidx_v1, sidx_v2)
        xv_v = (xv_v0, xv_v1, xv_v2)
        cid = lax.axis_index("c")
        sid = lax.axis_index("s")
        slot = sid // tiles_per_item
        half = sid % tiles_per_item
        item = cid * items_per_core + slot

        gbase = item * IN_DIM
        sbase = slot * OUT_DIM
        nnz_base = half * nnz_per_tile

        def dma_cr(ci):
            b = ci % 2
            base = nnz_base + ci * CHUNK
            return (
                pltpu.async_copy(cols_hbm.at[item, pl.ds(base, CHUNK)],
                                 cols_v[b], sem_cr),
                pltpu.async_copy(rows_hbm.at[item, pl.ds(base, CHUNK)],
                                 rows_v[b], sem_cr),
            )

        def dma_v(ci):
            base = nnz_base + ci * CHUNK
            return pltpu.async_copy(vals_hbm.at[item, pl.ds(base, CHUNK)],
                                    vals_v[ci % 3], sem_v)

        cr_pend = {0: dma_cr(0), 1: dma_cr(1)}
        v_pend = {0: dma_v(0), 1: dma_v(1)}

        @plsc.parallel_loop(0, CHUNK, step=L, unroll=8)
        def zero_body(i):
            xv_v0[pl.ds(i, L)] = jnp.zeros((L,), jnp.float32)
        z_left = out_slice
        z_off = slot * OUT_DIM + half * out_slice
        while z_left > 0:
            z = min(z_left, CHUNK)
            pltpu.sync_copy(xv_v0.at[pl.ds(0, z)],
                            shared_acc.at[pl.ds(z_off, z)])
            z_off += z
            z_left -= z

        plsc.subcore_barrier()

        def remap(b2, b3, i):
            sl = pl.ds(i, L)
            cc = cols_v[b2][sl]
            q = _floordiv3(cc)
            gidx_v[b2][sl] = cc * (H * W) - q * (3 * H * W - 1) + gbase
            rr = rows_v[b2][sl]
            q2 = _floordiv3(rr)
            sidx_v[b3][sl] = (
                rr * (OUT_H * OUT_W) - q2 * (3 * OUT_H * OUT_W - 1) + sbase)

        gats, scat = {}, {}
        for k in (0, 1):
            for d in cr_pend.pop(k):
                d.wait()

            @plsc.parallel_loop(0, CHUNK, step=L, unroll=8)
            def pre_remap(i):
                remap(k % 2, k % 3, i)

            gats[k] = pltpu.async_copy(x_hbm.at[gidx_v[k % 2]],
                                       xv_v[k % 3], sem_g)
            if k + 2 < n_chunks:
                cr_pend[k + 2] = dma_cr(k + 2)
        if 2 < n_chunks:
            v_pend[2] = dma_v(2)

        for k in range(n_chunks):
            b2 = k % 2
            b3 = k % 3
            if k + 2 < n_chunks:
                for d in cr_pend.pop(k + 2):
                    d.wait()
            if k in v_pend:
                v_pend.pop(k).wait()
            gats.pop(k).wait()
            if k - 1 in scat:
                scat.pop(k - 1).wait()

            if k + 2 < n_chunks:
                nb3 = (k + 2) % 3

                @plsc.parallel_loop(0, CHUNK, step=L, unroll=8)
                def fused_body(i):
                    sl = pl.ds(i, L)
                    xv_v[b3][sl] = xv_v[b3][sl] * vals_v[b3][sl]
                    remap(b2, nb3, i)

                gats[k + 2] = pltpu.async_copy(x_hbm.at[gidx_v[b2]],
                                               xv_v[nb3], sem_g)
            else:
                @plsc.parallel_loop(0, CHUNK, step=L, unroll=8)
                def tail_body(i):
                    sl = pl.ds(i, L)
                    xv_v[b3][sl] = xv_v[b3][sl] * vals_v[b3][sl]

            if k + 4 < n_chunks:
                cr_pend[k + 4] = dma_cr(k + 4)
            if k + 3 < n_chunks:
                v_pend[k + 3] = dma_v(k + 3)

            scat[k] = pltpu.async_copy(
                xv_v[b3], shared_acc.at[sidx_v[b3]], sem_s, add=True)

        scat.pop(n_chunks - 1).wait()

        plsc.subcore_barrier()

        pltpu.sync_copy(
            shared_acc.at[pl.ds(slot * OUT_DIM + half * out_slice, out_slice)],
            out_hbm.at[pl.ds(item * OUT_DIM + half * out_slice, out_slice)])

    return run(x1d, rows, cols, vals)


def kernel(x, rows, cols, vals, mask):
    x1d = x.reshape(B * IN_DIM)
    out = _sc_spmv(x1d, rows, cols, vals)
    result = out.reshape(B, C, OUT_H, OUT_W)
    zero = (x1d[0] - x1d[0]).astype(jnp.float32)
    masks = jnp.transpose(mask, (0, 3, 1, 2)) + zero
    return (result, masks)
